# baseline (speedup 1.0000x reference)
import functools
import math

import jax
import jax.numpy as jnp
from jax import lax
from jax.experimental import pallas as pl
from jax.experimental.pallas import tpu as pltpu
from jax.experimental.pallas import tpu_sc as plsc

D_MODEL = 128
SCALE = math.sqrt(D_MODEL)

_info = plsc.get_sparse_core_info()
NC, NS, L = _info.num_cores, _info.num_subcores, _info.num_lanes
NW = NC * NS

R = 2
NBUF = 8
LA = 3


def _make_kernel(n_rows, seq):
    assert n_rows % NW == 0
    rows_per_w = n_rows // NW
    assert rows_per_w % R == 0
    chunks_per_w = rows_per_w // R
    assert chunks_per_w % NBUF == 0
    mesh = plsc.VectorSubcoreMesh(core_axis_name="c", subcore_axis_name="s")

    @functools.partial(
        pl.kernel,
        mesh=mesh,
        out_type=jax.ShapeDtypeStruct((n_rows, seq, D_MODEL), jnp.float32),
        scratch_types=(
            [pltpu.VMEM((rows_per_w, seq), jnp.int32)]
            + [pltpu.VMEM((R, seq, D_MODEL), jnp.float32)] * NBUF
            + [pltpu.SemaphoreType.DMA] * (2 * NBUF)
        ),
    )
    def k(x_hbm, table_hbm, out_hbm, idx_v, *rest):
        bufs = rest[:NBUF]
        gsems = rest[NBUF:2 * NBUF]
        osems = rest[2 * NBUF:3 * NBUF]
        wid = lax.axis_index("s") * NC + lax.axis_index("c")
        row0 = wid * rows_per_w
        pltpu.sync_copy(x_hbm.at[pl.ds(row0, rows_per_w)], idx_v)

        def issue_gather(g, b):
            for r in range(R):
                pltpu.async_copy(
                    table_hbm.at[idx_v.at[g * R + r]], bufs[b].at[r],
                    gsems[b])

        def wait_gather(b):
            pltpu.make_async_copy(
                out_hbm.at[pl.ds(0, R)], bufs[b], gsems[b]).wait()

        def wait_out(b):
            pltpu.make_async_copy(
                bufs[b], out_hbm.at[pl.ds(0, R)], osems[b]).wait()

        for b in range(LA):
            issue_gather(b, b)

        def scale_buf(buf):
            def scale_col(s, c):
                for r in range(R):
                    for j in range(D_MODEL // L):
                        buf[r, s, pl.ds(j * L, L)] = (
                            buf[r, s, pl.ds(j * L, L)] * SCALE)
                return c
            lax.fori_loop(0, seq, scale_col, 0)

        def outer(g2, carry):
            for b in range(NBUF):
                g = g2 * NBUF + b
                bl = (b + LA) % NBUF
                gl = g + LA

                @pl.when(gl < chunks_per_w)
                def _issue():
                    @pl.when(gl >= NBUF)
                    def _wait_out():
                        wait_out(bl)
                    issue_gather(gl, bl)

                wait_gather(b)
                scale_buf(bufs[b])
                pltpu.async_copy(
                    bufs[b], out_hbm.at[pl.ds(row0 + g * R, R)], osems[b])
            return carry

        lax.fori_loop(0, chunks_per_w // NBUF, outer, 0)

        for b in range(NBUF):
            wait_out(b)

    return k


@jax.jit
def kernel(x, table):
    n_rows, seq = x.shape
    return _make_kernel(n_rows, seq)(x.astype(jnp.int32), table)


# speedup vs baseline: 1.0021x; 1.0021x over previous
import functools
import math

import jax
import jax.numpy as jnp
from jax import lax
from jax.experimental import pallas as pl
from jax.experimental.pallas import tpu as pltpu
from jax.experimental.pallas import tpu_sc as plsc

D_MODEL = 128
SCALE = math.sqrt(D_MODEL)

_info = plsc.get_sparse_core_info()
NC, NS, L = _info.num_cores, _info.num_subcores, _info.num_lanes
NW = NC * NS

R = 2
NBUF = 8
LA = 3


def _make_kernel(n_rows, seq):
    assert n_rows % NW == 0
    rows_per_w = n_rows // NW
    assert rows_per_w % R == 0
    chunks_per_w = rows_per_w // R
    assert chunks_per_w % NBUF == 0
    mesh = plsc.VectorSubcoreMesh(core_axis_name="c", subcore_axis_name="s")

    @functools.partial(
        pl.kernel,
        mesh=mesh,
        out_type=jax.ShapeDtypeStruct((n_rows, seq, D_MODEL), jnp.float32),
        compiler_params=pltpu.CompilerParams(use_tc_tiling_on_sc=True),
        scratch_types=(
            [pltpu.VMEM((rows_per_w, seq), jnp.int32)]
            + [pltpu.VMEM((R, seq, D_MODEL), jnp.float32)] * NBUF
            + [pltpu.SemaphoreType.DMA] * (2 * NBUF)
        ),
    )
    def k(x_hbm, table_hbm, out_hbm, idx_v, *rest):
        bufs = rest[:NBUF]
        gsems = rest[NBUF:2 * NBUF]
        osems = rest[2 * NBUF:3 * NBUF]
        wid = lax.axis_index("s") * NC + lax.axis_index("c")
        row0 = wid * rows_per_w
        pltpu.sync_copy(x_hbm.at[pl.ds(row0, rows_per_w)], idx_v)

        def issue_gather(g, b):
            for r in range(R):
                pltpu.async_copy(
                    table_hbm.at[idx_v.at[g * R + r]], bufs[b].at[r],
                    gsems[b])

        def wait_gather(b):
            pltpu.make_async_copy(
                out_hbm.at[pl.ds(0, R)], bufs[b], gsems[b]).wait()

        def wait_out(b):
            pltpu.make_async_copy(
                bufs[b], out_hbm.at[pl.ds(0, R)], osems[b]).wait()

        for b in range(LA):
            issue_gather(b, b)

        def scale_buf(buf):
            def scale_col(s, c):
                for r in range(R):
                    for j in range(D_MODEL // L):
                        buf[r, s, pl.ds(j * L, L)] = (
                            buf[r, s, pl.ds(j * L, L)] * SCALE)
                return c
            lax.fori_loop(0, seq, scale_col, 0)

        def outer(g2, carry):
            for b in range(NBUF):
                g = g2 * NBUF + b
                bl = (b + LA) % NBUF
                gl = g + LA

                @pl.when(gl < chunks_per_w)
                def _issue():
                    @pl.when(gl >= NBUF)
                    def _wait_out():
                        wait_out(bl)
                    issue_gather(gl, bl)

                wait_gather(b)
                scale_buf(bufs[b])
                pltpu.async_copy(
                    bufs[b], out_hbm.at[pl.ds(row0 + g * R, R)], osems[b])
            return carry

        lax.fori_loop(0, chunks_per_w // NBUF, outer, 0)

        for b in range(NBUF):
            wait_out(b)

    return k


@jax.jit
def kernel(x, table):
    n_rows, seq = x.shape
    return _make_kernel(n_rows, seq)(x.astype(jnp.int32), table)
